# baseline (speedup 1.0000x reference)
import functools
import math

import jax
import jax.numpy as jnp
from jax import lax
from jax.experimental import pallas as pl
from jax.experimental.pallas import tpu as pltpu
from jax.experimental.pallas import tpu_sc as plsc

B, L = 4096, 200
D_HALF = 64
VOCAB = 100000
TW = 4 * D_HALF
N = B * L
NC, NS = 2, 16
NW = NC * NS
PER_W = N // NW
CHUNK = 128
N_CHUNKS = PER_W // CHUNK


def _sc_body(x_hbm, tbl_hbm, out_hbm, idx_v, rows_v, out_v, sem):
    wid = lax.axis_index("s") * NC + lax.axis_index("c")
    wbase = wid * PER_W

    def chunk_body(ci, carry):
        base = wbase + ci * CHUNK
        pltpu.sync_copy(x_hbm.at[pl.ds(base, CHUNK)], idx_v)
        pltpu.async_copy(tbl_hbm.at[idx_v], rows_v, sem).wait()

        def row_body(i, carry2):
            g = base + i
            pos = (lax.rem(g, L) + 1).astype(jnp.float32)
            pv = jnp.full((16,), pos, jnp.float32)
            for j in range(D_HALF // 16):
                sl = pl.ds(j * 16, 16)
                sl_f = pl.ds(D_HALF + j * 16, 16)
                sl_b = pl.ds(2 * D_HALF + j * 16, 16)
                out_v[i, sl] = rows_v[i, sl]
                out_v[i, sl_f] = pv * rows_v[i, sl_f] + rows_v[i, sl_b]
            return carry2

        lax.fori_loop(0, CHUNK, row_body, 0, unroll=False)
        pltpu.sync_copy(out_v, out_hbm.at[pl.ds(base, CHUNK)])
        return carry

    lax.fori_loop(0, N_CHUNKS, chunk_body, 0, unroll=False)


@functools.cache
def _sc_gather_phase():
    return pl.kernel(
        _sc_body,
        mesh=plsc.VectorSubcoreMesh(core_axis_name="c", subcore_axis_name="s"),
        out_type=jax.ShapeDtypeStruct((N, 2 * D_HALF), jnp.float32),
        scratch_types=[
            pltpu.VMEM((CHUNK,), jnp.int32),
            pltpu.VMEM((CHUNK, TW), jnp.float32),
            pltpu.VMEM((CHUNK, 2 * D_HALF), jnp.float32),
            pltpu.SemaphoreType.DMA,
        ],
    )


ROWS_PER_BLK = 1024


def _tc_trig_body(ap_ref, out_ref):
    amp = ap_ref[:, 0:D_HALF]
    ph = ap_ref[:, D_HALF:2 * D_HALF]
    out_ref[:, 0:D_HALF] = amp * jnp.cos(ph)
    out_ref[:, D_HALF:2 * D_HALF] = amp * jnp.sin(ph)


def _tc_trig(ap):
    return pl.pallas_call(
        _tc_trig_body,
        grid=(N // ROWS_PER_BLK,),
        in_specs=[pl.BlockSpec((ROWS_PER_BLK, 2 * D_HALF), lambda i: (i, 0))],
        out_specs=pl.BlockSpec((ROWS_PER_BLK, 2 * D_HALF), lambda i: (i, 0)),
        out_shape=jax.ShapeDtypeStruct((N, 2 * D_HALF), jnp.float32),
    )(ap)


def kernel(x, word_table, freq_table, phase_table):
    x_flat = x.reshape(N)
    tbl = jnp.concatenate(
        [word_table, freq_table, phase_table,
         jnp.zeros((VOCAB, D_HALF), jnp.float32)], axis=1)
    ap = _sc_gather_phase()(x_flat, tbl)
    out = _tc_trig(ap)
    return out.reshape(B, L, 2 * D_HALF)


# speedup vs baseline: 1.0311x; 1.0311x over previous
import functools
import math

import jax
import jax.numpy as jnp
from jax import lax
from jax.experimental import pallas as pl
from jax.experimental.pallas import tpu as pltpu
from jax.experimental.pallas import tpu_sc as plsc

B, L = 4096, 200
D_HALF = 64
N = B * L
NC, NS = 2, 16
NW = NC * NS
PER_W = N // NW
CHUNK = 128
N_CHUNKS = PER_W // CHUNK


def _sc_body(x_hbm, word_hbm, freq_hbm, phase_hbm, out_hbm,
             idx_v, amp_v, f_v, b_v, out_v, sem_a, sem_f, sem_b):
    wid = lax.axis_index("s") * NC + lax.axis_index("c")
    wbase = wid * PER_W

    def chunk_body(ci, carry):
        base = wbase + ci * CHUNK
        pltpu.sync_copy(x_hbm.at[pl.ds(base, CHUNK)], idx_v)
        ca = pltpu.async_copy(word_hbm.at[idx_v], amp_v, sem_a)
        cf = pltpu.async_copy(freq_hbm.at[idx_v], f_v, sem_f)
        cb = pltpu.async_copy(phase_hbm.at[idx_v], b_v, sem_b)
        ca.wait()
        cf.wait()
        cb.wait()

        def row_body(i, carry2):
            g = base + i
            pos = (lax.rem(g, L) + 1).astype(jnp.float32)
            pv = jnp.full((16,), pos, jnp.float32)
            for j in range(D_HALF // 16):
                sl = pl.ds(j * 16, 16)
                sl_o = pl.ds(D_HALF + j * 16, 16)
                out_v[i, sl] = amp_v[i, sl]
                out_v[i, sl_o] = pv * f_v[i, sl] + b_v[i, sl]
            return carry2

        lax.fori_loop(0, CHUNK, row_body, 0, unroll=False)
        pltpu.sync_copy(out_v, out_hbm.at[pl.ds(base, CHUNK)])
        return carry

    lax.fori_loop(0, N_CHUNKS, chunk_body, 0, unroll=False)


@functools.cache
def _sc_gather_phase():
    return pl.kernel(
        _sc_body,
        mesh=plsc.VectorSubcoreMesh(core_axis_name="c", subcore_axis_name="s"),
        compiler_params=pltpu.CompilerParams(use_tc_tiling_on_sc=False),
        out_type=jax.ShapeDtypeStruct((N, 2 * D_HALF), jnp.float32),
        scratch_types=[
            pltpu.VMEM((CHUNK,), jnp.int32),
            pltpu.VMEM((CHUNK, D_HALF), jnp.float32),
            pltpu.VMEM((CHUNK, D_HALF), jnp.float32),
            pltpu.VMEM((CHUNK, D_HALF), jnp.float32),
            pltpu.VMEM((CHUNK, 2 * D_HALF), jnp.float32),
            pltpu.SemaphoreType.DMA,
            pltpu.SemaphoreType.DMA,
            pltpu.SemaphoreType.DMA,
        ],
    )


ROWS_PER_BLK = 1024


def _tc_trig_body(ap_ref, out_ref):
    amp = ap_ref[:, 0:D_HALF]
    ph = ap_ref[:, D_HALF:2 * D_HALF]
    out_ref[:, 0:D_HALF] = amp * jnp.cos(ph)
    out_ref[:, D_HALF:2 * D_HALF] = amp * jnp.sin(ph)


def _tc_trig(ap):
    return pl.pallas_call(
        _tc_trig_body,
        grid=(N // ROWS_PER_BLK,),
        in_specs=[pl.BlockSpec((ROWS_PER_BLK, 2 * D_HALF), lambda i: (i, 0))],
        out_specs=pl.BlockSpec((ROWS_PER_BLK, 2 * D_HALF), lambda i: (i, 0)),
        out_shape=jax.ShapeDtypeStruct((N, 2 * D_HALF), jnp.float32),
    )(ap)


def kernel(x, word_table, freq_table, phase_table):
    x_flat = x.reshape(N)
    ap = _sc_gather_phase()(x_flat, word_table, freq_table, phase_table)
    out = _tc_trig(ap)
    return out.reshape(B, L, 2 * D_HALF)


# speedup vs baseline: 1.3519x; 1.3112x over previous
import functools
import math

import jax
import jax.numpy as jnp
from jax import lax
from jax.experimental import pallas as pl
from jax.experimental.pallas import tpu as pltpu
from jax.experimental.pallas import tpu_sc as plsc

B, L = 4096, 200
D_HALF = 64
N = B * L
NC, NS = 2, 16
NW = NC * NS
PER_W = N // NW
CHUNK = 256
SUB = 128
N_CHUNKS = PER_W // CHUNK
OUTW = 4 * D_HALF


def _sc_body(x_hbm, word_hbm, freq_hbm, phase_hbm, out_hbm,
             idx_a, amp_a, f_a, b_a, idx_b, amp_b, f_b, b_b,
             sg_a, sw_a, sg_b, sw_b):
    wid = lax.axis_index("s") * NC + lax.axis_index("c")
    wbase = wid * PER_W

    sets = ((idx_a, amp_a, f_a, b_a, sg_a, sw_a),
            (idx_b, amp_b, f_b, b_b, sg_b, sw_b))

    def do_chunk(ci, idx_v, amp_v, f_v, b_v, sg, sw):
        base = wbase + ci * CHUNK
        @pl.when(ci >= 2)
        def _():
            pltpu.make_async_copy(
                amp_v, out_hbm.at[pl.ds(base, CHUNK), pl.ds(0, D_HALF)], sw
            ).wait()
            pltpu.make_async_copy(
                f_v, out_hbm.at[pl.ds(base, CHUNK), pl.ds(D_HALF, D_HALF)], sw
            ).wait()
            pltpu.make_async_copy(
                b_v, out_hbm.at[pl.ds(base, CHUNK), pl.ds(2 * D_HALF, D_HALF)], sw
            ).wait()

        pltpu.sync_copy(x_hbm.at[pl.ds(base, CHUNK)], idx_v)
        for k in range(CHUNK // SUB):
            sl = pl.ds(k * SUB, SUB)
            pltpu.async_copy(word_hbm.at[idx_v.at[sl]], amp_v.at[sl], sg)
            pltpu.async_copy(freq_hbm.at[idx_v.at[sl]], f_v.at[sl], sg)
            pltpu.async_copy(phase_hbm.at[idx_v.at[sl]], b_v.at[sl], sg)
        for k in range(CHUNK // SUB):
            sl = pl.ds(k * SUB, SUB)
            pltpu.make_async_copy(word_hbm.at[idx_v.at[sl]], amp_v.at[sl], sg).wait()
            pltpu.make_async_copy(freq_hbm.at[idx_v.at[sl]], f_v.at[sl], sg).wait()
            pltpu.make_async_copy(phase_hbm.at[idx_v.at[sl]], b_v.at[sl], sg).wait()
        pltpu.async_copy(
            amp_v, out_hbm.at[pl.ds(base, CHUNK), pl.ds(0, D_HALF)], sw)
        pltpu.async_copy(
            f_v, out_hbm.at[pl.ds(base, CHUNK), pl.ds(D_HALF, D_HALF)], sw)
        pltpu.async_copy(
            b_v, out_hbm.at[pl.ds(base, CHUNK), pl.ds(2 * D_HALF, D_HALF)], sw)

    def pair_body(g, carry):
        do_chunk(2 * g, *sets[0])
        do_chunk(2 * g + 1, *sets[1])
        return carry

    lax.fori_loop(0, N_CHUNKS // 2, pair_body, 0, unroll=False)

    for (idx_v, amp_v, f_v, b_v, sg, sw) in sets:
        pltpu.make_async_copy(
            amp_v, out_hbm.at[pl.ds(wbase, CHUNK), pl.ds(0, D_HALF)], sw).wait()
        pltpu.make_async_copy(
            f_v, out_hbm.at[pl.ds(wbase, CHUNK), pl.ds(D_HALF, D_HALF)], sw).wait()
        pltpu.make_async_copy(
            b_v, out_hbm.at[pl.ds(wbase, CHUNK), pl.ds(2 * D_HALF, D_HALF)], sw).wait()


@functools.cache
def _sc_gather():
    buf = lambda: pltpu.VMEM((CHUNK, D_HALF), jnp.float32)
    return pl.kernel(
        _sc_body,
        mesh=plsc.VectorSubcoreMesh(core_axis_name="c", subcore_axis_name="s"),
        compiler_params=pltpu.CompilerParams(use_tc_tiling_on_sc=False),
        out_type=jax.ShapeDtypeStruct((N, OUTW), jnp.float32),
        scratch_types=[
            pltpu.VMEM((CHUNK,), jnp.int32), buf(), buf(), buf(),
            pltpu.VMEM((CHUNK,), jnp.int32), buf(), buf(), buf(),
            pltpu.SemaphoreType.DMA, pltpu.SemaphoreType.DMA,
            pltpu.SemaphoreType.DMA, pltpu.SemaphoreType.DMA,
        ],
    )


ROWS_PER_BLK = 1024

_MAGIC = 12582912.0
_INV_PIO2 = 0.6366197723675814
_PIO2_HI = 1.57080078125
_PIO2_MID = -4.454455029158992e-06
_S1, _S2 = -0.16666667, 0.008332161
_C1, _C2, _C3 = -0.5, 0.041666418, -0.0013889048


def _sincos(ph):
    t = ph * _INV_PIO2
    n_big = t + _MAGIC
    nf = n_big - _MAGIC
    r = ph - nf * _PIO2_HI
    r = r - nf * _PIO2_MID
    r2 = r * r
    s = r + r * r2 * (_S1 + r2 * _S2)
    c = 1.0 + r2 * (_C1 + r2 * (_C2 + r2 * _C3))
    ni = lax.bitcast_convert_type(n_big, jnp.int32)
    swap = (ni & 1) == 1
    sinv = jnp.where(swap, c, s)
    cosv = jnp.where(swap, s, c)
    sgn_s = (ni & 2) << 30
    sgn_c = ((ni + 1) & 2) << 30
    sinv = lax.bitcast_convert_type(
        lax.bitcast_convert_type(sinv, jnp.int32) ^ sgn_s, jnp.float32)
    cosv = lax.bitcast_convert_type(
        lax.bitcast_convert_type(cosv, jnp.int32) ^ sgn_c, jnp.float32)
    return sinv, cosv


def _tc_trig_body(packed_ref, out_ref):
    i = pl.program_id(0)
    amp = packed_ref[:, 0:D_HALF]
    f = packed_ref[:, D_HALF:2 * D_HALF]
    b = packed_ref[:, 2 * D_HALF:3 * D_HALF]
    row = lax.broadcasted_iota(jnp.int32, (ROWS_PER_BLK, 1), 0) + i * ROWS_PER_BLK
    pos = (lax.rem(row, L) + 1).astype(jnp.float32)
    ph = pos * f + b
    sinv, cosv = _sincos(ph)
    out_ref[:, 0:D_HALF] = amp * cosv
    out_ref[:, D_HALF:2 * D_HALF] = amp * sinv


def _tc_trig(packed):
    return pl.pallas_call(
        _tc_trig_body,
        grid=(N // ROWS_PER_BLK,),
        in_specs=[
            pl.BlockSpec((ROWS_PER_BLK, OUTW), lambda i: (i, 0)),
        ],
        out_specs=pl.BlockSpec((ROWS_PER_BLK, 2 * D_HALF), lambda i: (i, 0)),
        out_shape=jax.ShapeDtypeStruct((N, 2 * D_HALF), jnp.float32),
    )(packed)


def kernel(x, word_table, freq_table, phase_table):
    x_flat = x.reshape(N)
    packed = _sc_gather()(x_flat, word_table, freq_table, phase_table)
    out = _tc_trig(packed)
    return out.reshape(B, L, 2 * D_HALF)


# speedup vs baseline: 1.9706x; 1.4576x over previous
import functools
import math

import jax
import jax.numpy as jnp
from jax import lax
from jax.experimental import pallas as pl
from jax.experimental.pallas import tpu as pltpu
from jax.experimental.pallas import tpu_sc as plsc

B, L = 4096, 200
D_HALF = 64
N = B * L
NC, NS = 2, 16
NW = NC * NS
PER_W = N // NW
CHUNK = 128
N_CHUNKS = PER_W // CHUNK


def _sc_body(x_hbm, wf_hbm, phase_hbm, wf_out, b_out,
             idx_a, wf_a, b_a, idx_b, wf_b, b_b,
             sg_a, sw_a, sg_b, sw_b):
    wid = lax.axis_index("s") * NC + lax.axis_index("c")
    wbase = wid * PER_W

    sets = ((idx_a, wf_a, b_a, sg_a, sw_a),
            (idx_b, wf_b, b_b, sg_b, sw_b))

    def do_chunk(ci, idx_v, wf_v, b_v, sg, sw):
        base = wbase + ci * CHUNK
        @pl.when(ci >= 2)
        def _():
            pltpu.make_async_copy(
                wf_v, wf_out.at[pl.ds(base, CHUNK)], sw).wait()
            pltpu.make_async_copy(
                b_v, b_out.at[pl.ds(base, CHUNK), pl.ds(0, D_HALF)], sw).wait()

        pltpu.sync_copy(x_hbm.at[pl.ds(base, CHUNK)], idx_v)
        pltpu.async_copy(wf_hbm.at[idx_v], wf_v, sg)
        pltpu.async_copy(phase_hbm.at[idx_v], b_v, sg)
        pltpu.make_async_copy(wf_hbm.at[idx_v], wf_v, sg).wait()
        pltpu.make_async_copy(phase_hbm.at[idx_v], b_v, sg).wait()
        pltpu.async_copy(wf_v, wf_out.at[pl.ds(base, CHUNK)], sw)
        pltpu.async_copy(b_v, b_out.at[pl.ds(base, CHUNK), pl.ds(0, D_HALF)], sw)

    def pair_body(g, carry):
        do_chunk(2 * g, *sets[0])
        do_chunk(2 * g + 1, *sets[1])
        return carry

    lax.fori_loop(0, N_CHUNKS // 2, pair_body, 0, unroll=False)

    for (idx_v, wf_v, b_v, sg, sw) in sets:
        pltpu.make_async_copy(wf_v, wf_out.at[pl.ds(wbase, CHUNK)], sw).wait()
        pltpu.make_async_copy(
            b_v, b_out.at[pl.ds(wbase, CHUNK), pl.ds(0, D_HALF)], sw).wait()


@functools.cache
def _sc_gather():
    return pl.kernel(
        _sc_body,
        mesh=plsc.VectorSubcoreMesh(core_axis_name="c", subcore_axis_name="s"),
        compiler_params=pltpu.CompilerParams(use_tc_tiling_on_sc=False),
        out_type=[
            jax.ShapeDtypeStruct((N, 2 * D_HALF), jnp.float32),
            jax.ShapeDtypeStruct((N, 2 * D_HALF), jnp.float32),
        ],
        scratch_types=[
            pltpu.VMEM((CHUNK,), jnp.int32),
            pltpu.VMEM((CHUNK, 2 * D_HALF), jnp.float32),
            pltpu.VMEM((CHUNK, D_HALF), jnp.float32),
            pltpu.VMEM((CHUNK,), jnp.int32),
            pltpu.VMEM((CHUNK, 2 * D_HALF), jnp.float32),
            pltpu.VMEM((CHUNK, D_HALF), jnp.float32),
            pltpu.SemaphoreType.DMA, pltpu.SemaphoreType.DMA,
            pltpu.SemaphoreType.DMA, pltpu.SemaphoreType.DMA,
        ],
    )


ROWS_PER_BLK = 1024

_MAGIC = 12582912.0
_INV_PIO2 = 0.6366197723675814
_PIO2_HI = 1.57080078125
_PIO2_MID = -4.454455029158992e-06
_S1, _S2 = -0.16666667, 0.008332161
_C1, _C2, _C3 = -0.5, 0.041666418, -0.0013889048


def _sincos(ph):
    t = ph * _INV_PIO2
    n_big = t + _MAGIC
    nf = n_big - _MAGIC
    r = ph - nf * _PIO2_HI
    r = r - nf * _PIO2_MID
    r2 = r * r
    s = r + r * r2 * (_S1 + r2 * _S2)
    c = 1.0 + r2 * (_C1 + r2 * (_C2 + r2 * _C3))
    ni = lax.bitcast_convert_type(n_big, jnp.int32)
    swap = (ni & 1) == 1
    sinv = jnp.where(swap, c, s)
    cosv = jnp.where(swap, s, c)
    sgn_s = (ni & 2) << 30
    sgn_c = ((ni + 1) & 2) << 30
    sinv = lax.bitcast_convert_type(
        lax.bitcast_convert_type(sinv, jnp.int32) ^ sgn_s, jnp.float32)
    cosv = lax.bitcast_convert_type(
        lax.bitcast_convert_type(cosv, jnp.int32) ^ sgn_c, jnp.float32)
    return sinv, cosv


def _tc_trig_body(wf_ref, b_ref, out_ref):
    i = pl.program_id(0)
    amp = wf_ref[:, 0:D_HALF]
    f = wf_ref[:, D_HALF:2 * D_HALF]
    b = b_ref[:, 0:D_HALF]
    row = lax.broadcasted_iota(jnp.int32, (ROWS_PER_BLK, 1), 0) + i * ROWS_PER_BLK
    pos = (lax.rem(row, L) + 1).astype(jnp.float32)
    ph = pos * f + b
    sinv, cosv = _sincos(ph)
    out_ref[:, 0:D_HALF] = amp * cosv
    out_ref[:, D_HALF:2 * D_HALF] = amp * sinv


def _tc_trig(wf, bx):
    return pl.pallas_call(
        _tc_trig_body,
        grid=(N // ROWS_PER_BLK,),
        in_specs=[
            pl.BlockSpec((ROWS_PER_BLK, 2 * D_HALF), lambda i: (i, 0)),
            pl.BlockSpec((ROWS_PER_BLK, 2 * D_HALF), lambda i: (i, 0)),
        ],
        out_specs=pl.BlockSpec((ROWS_PER_BLK, 2 * D_HALF), lambda i: (i, 0)),
        out_shape=jax.ShapeDtypeStruct((N, 2 * D_HALF), jnp.float32),
    )(wf, bx)


def kernel(x, word_table, freq_table, phase_table):
    x_flat = x.reshape(N)
    wf_tbl = jnp.concatenate([word_table, freq_table], axis=1)
    wf, bx = _sc_gather()(x_flat, wf_tbl, phase_table)
    out = _tc_trig(wf, bx)
    return out.reshape(B, L, 2 * D_HALF)


# speedup vs baseline: 2.4213x; 1.2288x over previous
import functools
import math

import jax
import jax.numpy as jnp
from jax import lax
from jax.experimental import pallas as pl
from jax.experimental.pallas import tpu as pltpu
from jax.experimental.pallas import tpu_sc as plsc

B, L = 4096, 200
D_HALF = 64
N = B * L
NC, NS = 2, 16
NW = NC * NS
PER_W = N // NW
CHUNK = 128
N_CHUNKS = PER_W // CHUNK


def _sc_body(x_hbm, wf_hbm, phase_hbm, wf_out, b_out,
             idx_a, wf_a, b_a, idx_b, wf_b, b_b,
             sg_a, sw_a, sg_b, sw_b):
    wid = lax.axis_index("s") * NC + lax.axis_index("c")
    wbase = wid * PER_W

    sets = ((idx_a, wf_a, b_a, sg_a, sw_a),
            (idx_b, wf_b, b_b, sg_b, sw_b))

    def do_chunk(ci, idx_v, wf_v, b_v, sg, sw):
        base = wbase + ci * CHUNK
        @pl.when(ci >= 2)
        def _():
            pltpu.make_async_copy(
                wf_v, wf_out.at[pl.ds(base, CHUNK)], sw).wait()
            pltpu.make_async_copy(
                b_v, b_out.at[pl.ds(base, CHUNK), pl.ds(0, D_HALF)], sw).wait()

        pltpu.sync_copy(x_hbm.at[pl.ds(base, CHUNK)], idx_v)
        pltpu.async_copy(wf_hbm.at[idx_v], wf_v, sg)
        pltpu.async_copy(phase_hbm.at[idx_v], b_v, sg)
        pltpu.make_async_copy(wf_hbm.at[idx_v], wf_v, sg).wait()
        pltpu.make_async_copy(phase_hbm.at[idx_v], b_v, sg).wait()
        pltpu.async_copy(wf_v, wf_out.at[pl.ds(base, CHUNK)], sw)
        pltpu.async_copy(b_v, b_out.at[pl.ds(base, CHUNK), pl.ds(0, D_HALF)], sw)

    def pair_body(g, carry):
        do_chunk(2 * g, *sets[0])
        do_chunk(2 * g + 1, *sets[1])
        return carry

    lax.fori_loop(0, N_CHUNKS // 2, pair_body, 0, unroll=False)

    for (idx_v, wf_v, b_v, sg, sw) in sets:
        pltpu.make_async_copy(wf_v, wf_out.at[pl.ds(wbase, CHUNK)], sw).wait()
        pltpu.make_async_copy(
            b_v, b_out.at[pl.ds(wbase, CHUNK), pl.ds(0, D_HALF)], sw).wait()


@functools.cache
def _sc_gather():
    return pl.kernel(
        _sc_body,
        mesh=plsc.VectorSubcoreMesh(core_axis_name="c", subcore_axis_name="s"),
        compiler_params=pltpu.CompilerParams(use_tc_tiling_on_sc=False),
        out_type=[
            jax.ShapeDtypeStruct((N, 2 * D_HALF), jnp.float32),
            jax.ShapeDtypeStruct((N, 2 * D_HALF), jnp.float32),
        ],
        scratch_types=[
            pltpu.VMEM((CHUNK,), jnp.int32),
            pltpu.VMEM((CHUNK, 2 * D_HALF), jnp.float32),
            pltpu.VMEM((CHUNK, D_HALF), jnp.float32),
            pltpu.VMEM((CHUNK,), jnp.int32),
            pltpu.VMEM((CHUNK, 2 * D_HALF), jnp.float32),
            pltpu.VMEM((CHUNK, D_HALF), jnp.float32),
            pltpu.SemaphoreType.DMA, pltpu.SemaphoreType.DMA,
            pltpu.SemaphoreType.DMA, pltpu.SemaphoreType.DMA,
        ],
    )


ROWS_PER_BLK = 2048

_MAGIC = 12582912.0
_INV_PIO2 = 0.6366197723675814
_PIO2_HI = 1.57080078125
_PIO2_MID = -4.454455029158992e-06
_S1, _S2 = -0.16666667, 0.008332161
_C1, _C2, _C3 = -0.5, 0.041666418, -0.0013889048


def _sincos(ph):
    t = ph * _INV_PIO2
    n_big = t + _MAGIC
    nf = n_big - _MAGIC
    r = ph - nf * _PIO2_HI
    r = r - nf * _PIO2_MID
    r2 = r * r
    s = r + r * r2 * (_S1 + r2 * _S2)
    c = 1.0 + r2 * (_C1 + r2 * (_C2 + r2 * _C3))
    ni = lax.bitcast_convert_type(n_big, jnp.int32)
    swap = (ni & 1) == 1
    sinv = jnp.where(swap, c, s)
    cosv = jnp.where(swap, s, c)
    sgn_s = (ni & 2) << 30
    sgn_c = ((ni + 1) & 2) << 30
    sinv = lax.bitcast_convert_type(
        lax.bitcast_convert_type(sinv, jnp.int32) ^ sgn_s, jnp.float32)
    cosv = lax.bitcast_convert_type(
        lax.bitcast_convert_type(cosv, jnp.int32) ^ sgn_c, jnp.float32)
    return sinv, cosv


def _pos_col(i, sub):
    half = ROWS_PER_BLK // 2
    off = lax.rem(i * ROWS_PER_BLK + sub * half, L)
    v = lax.broadcasted_iota(jnp.int32, (half, 1), 0) + off
    q = (v * 5243) >> 20
    return (v - q * L + 1).astype(jnp.float32)


def _tc_trig_body(wf_ref, b_ref, out_ref):
    i = pl.program_id(0)
    half = ROWS_PER_BLK // 2
    lo, hi = pl.ds(0, half), pl.ds(half, half)
    ph_lo = _pos_col(i, 0) * wf_ref[lo, D_HALF:2 * D_HALF] + b_ref[lo, 0:D_HALF]
    ph_hi = _pos_col(i, 1) * wf_ref[hi, D_HALF:2 * D_HALF] + b_ref[hi, 0:D_HALF]
    ph = jnp.concatenate([ph_lo, ph_hi], axis=1)
    sinv, cosv = _sincos(ph)
    amp_lo = wf_ref[lo, 0:D_HALF]
    amp_hi = wf_ref[hi, 0:D_HALF]
    out_ref[lo, 0:D_HALF] = amp_lo * cosv[:, 0:D_HALF]
    out_ref[lo, D_HALF:2 * D_HALF] = amp_lo * sinv[:, 0:D_HALF]
    out_ref[hi, 0:D_HALF] = amp_hi * cosv[:, D_HALF:2 * D_HALF]
    out_ref[hi, D_HALF:2 * D_HALF] = amp_hi * sinv[:, D_HALF:2 * D_HALF]


def _tc_trig(wf, bx):
    return pl.pallas_call(
        _tc_trig_body,
        grid=(N // ROWS_PER_BLK,),
        in_specs=[
            pl.BlockSpec((ROWS_PER_BLK, 2 * D_HALF), lambda i: (i, 0)),
            pl.BlockSpec((ROWS_PER_BLK, 2 * D_HALF), lambda i: (i, 0)),
        ],
        out_specs=pl.BlockSpec((ROWS_PER_BLK, 2 * D_HALF), lambda i: (i, 0)),
        out_shape=jax.ShapeDtypeStruct((N, 2 * D_HALF), jnp.float32),
    )(wf, bx)


def kernel(x, word_table, freq_table, phase_table):
    x_flat = x.reshape(N)
    wf_tbl = jnp.concatenate([word_table, freq_table], axis=1)
    wf, bx = _sc_gather()(x_flat, wf_tbl, phase_table)
    out = _tc_trig(wf, bx)
    return out.reshape(B, L, 2 * D_HALF)


# speedup vs baseline: 2.7352x; 1.1296x over previous
import functools
import math

import jax
import jax.numpy as jnp
from jax import lax
from jax.experimental import pallas as pl
from jax.experimental.pallas import tpu as pltpu
from jax.experimental.pallas import tpu_sc as plsc

B, L = 4096, 200
D_HALF = 64
N = B * L
NSPLIT = 2
NH = N // NSPLIT
NC, NS = 2, 16
NW = NC * NS
PER_W = NH // NW
CHUNK = 128
N_CHUNKS = PER_W // CHUNK


def _sc_body(x_hbm, wf_hbm, phase_hbm, wf_out, b_out,
             idx_a, wf_a, b_a, idx_b, wf_b, b_b,
             sg_a, sw_a, sg_b, sw_b):
    wid = lax.axis_index("s") * NC + lax.axis_index("c")
    wbase = wid * PER_W

    sets = ((idx_a, wf_a, b_a, sg_a, sw_a),
            (idx_b, wf_b, b_b, sg_b, sw_b))

    def do_chunk(ci, idx_v, wf_v, b_v, sg, sw):
        base = wbase + ci * CHUNK
        @pl.when(ci >= 2)
        def _():
            pltpu.make_async_copy(
                wf_v, wf_out.at[pl.ds(base, CHUNK)], sw).wait()
            pltpu.make_async_copy(
                b_v, b_out.at[pl.ds(base, CHUNK), pl.ds(0, D_HALF)], sw).wait()

        pltpu.sync_copy(x_hbm.at[pl.ds(base, CHUNK)], idx_v)
        pltpu.async_copy(wf_hbm.at[idx_v], wf_v, sg)
        pltpu.async_copy(phase_hbm.at[idx_v], b_v, sg)
        pltpu.make_async_copy(wf_hbm.at[idx_v], wf_v, sg).wait()
        pltpu.make_async_copy(phase_hbm.at[idx_v], b_v, sg).wait()
        pltpu.async_copy(wf_v, wf_out.at[pl.ds(base, CHUNK)], sw)
        pltpu.async_copy(b_v, b_out.at[pl.ds(base, CHUNK), pl.ds(0, D_HALF)], sw)

    def pair_body(g, carry):
        do_chunk(2 * g, *sets[0])
        do_chunk(2 * g + 1, *sets[1])
        return carry

    lax.fori_loop(0, N_CHUNKS // 2, pair_body, 0, unroll=False)

    for (idx_v, wf_v, b_v, sg, sw) in sets:
        pltpu.make_async_copy(wf_v, wf_out.at[pl.ds(wbase, CHUNK)], sw).wait()
        pltpu.make_async_copy(
            b_v, b_out.at[pl.ds(wbase, CHUNK), pl.ds(0, D_HALF)], sw).wait()


@functools.cache
def _sc_gather():
    return pl.kernel(
        _sc_body,
        mesh=plsc.VectorSubcoreMesh(core_axis_name="c", subcore_axis_name="s"),
        compiler_params=pltpu.CompilerParams(use_tc_tiling_on_sc=False),
        out_type=[
            jax.ShapeDtypeStruct((NH, 2 * D_HALF), jnp.float32),
            jax.ShapeDtypeStruct((NH, 2 * D_HALF), jnp.float32),
        ],
        scratch_types=[
            pltpu.VMEM((CHUNK,), jnp.int32),
            pltpu.VMEM((CHUNK, 2 * D_HALF), jnp.float32),
            pltpu.VMEM((CHUNK, D_HALF), jnp.float32),
            pltpu.VMEM((CHUNK,), jnp.int32),
            pltpu.VMEM((CHUNK, 2 * D_HALF), jnp.float32),
            pltpu.VMEM((CHUNK, D_HALF), jnp.float32),
            pltpu.SemaphoreType.DMA, pltpu.SemaphoreType.DMA,
            pltpu.SemaphoreType.DMA, pltpu.SemaphoreType.DMA,
        ],
    )


ROWS_PER_BLK = 2048
BLKS_PER_STAGE = NH // ROWS_PER_BLK

_MAGIC = 12582912.0
_INV_PIO2 = 0.6366197723675814
_PIO2_HI = 1.57080078125
_PIO2_MID = -4.454455029158992e-06
_S1, _S2 = -0.16666667, 0.008332161
_C1, _C2, _C3 = -0.5, 0.041666418, -0.0013889048


def _sincos(ph):
    t = ph * _INV_PIO2
    n_big = t + _MAGIC
    nf = n_big - _MAGIC
    r = ph - nf * _PIO2_HI
    r = r - nf * _PIO2_MID
    r2 = r * r
    s = r + r * r2 * (_S1 + r2 * _S2)
    c = 1.0 + r2 * (_C1 + r2 * (_C2 + r2 * _C3))
    ni = lax.bitcast_convert_type(n_big, jnp.int32)
    swap = (ni & 1) == 1
    sinv = jnp.where(swap, c, s)
    cosv = jnp.where(swap, s, c)
    sgn_s = (ni & 2) << 30
    sgn_c = ((ni + 1) & 2) << 30
    sinv = lax.bitcast_convert_type(
        lax.bitcast_convert_type(sinv, jnp.int32) ^ sgn_s, jnp.float32)
    cosv = lax.bitcast_convert_type(
        lax.bitcast_convert_type(cosv, jnp.int32) ^ sgn_c, jnp.float32)
    return sinv, cosv


def _pos_col(i, sub):
    half = ROWS_PER_BLK // 2
    off = lax.rem(i * ROWS_PER_BLK + sub * half, L)
    v = lax.broadcasted_iota(jnp.int32, (half, 1), 0) + off
    q = (v * 5243) >> 20
    return (v - q * L + 1).astype(jnp.float32)


def _tc_trig_body(wf_ref, b_ref, out_ref):
    i = pl.program_id(0)
    half = ROWS_PER_BLK // 2
    lo, hi = pl.ds(0, half), pl.ds(half, half)
    ph_lo = _pos_col(i, 0) * wf_ref[lo, D_HALF:2 * D_HALF] + b_ref[lo, 0:D_HALF]
    ph_hi = _pos_col(i, 1) * wf_ref[hi, D_HALF:2 * D_HALF] + b_ref[hi, 0:D_HALF]
    ph = jnp.concatenate([ph_lo, ph_hi], axis=1)
    sinv, cosv = _sincos(ph)
    amp_lo = wf_ref[lo, 0:D_HALF]
    amp_hi = wf_ref[hi, 0:D_HALF]
    out_ref[lo, 0:D_HALF] = amp_lo * cosv[:, 0:D_HALF]
    out_ref[lo, D_HALF:2 * D_HALF] = amp_lo * sinv[:, 0:D_HALF]
    out_ref[hi, 0:D_HALF] = amp_hi * cosv[:, D_HALF:2 * D_HALF]
    out_ref[hi, D_HALF:2 * D_HALF] = amp_hi * sinv[:, D_HALF:2 * D_HALF]


def _tc_trig_body_alias(wf_ref, b_ref, _prev_ref, out_ref):
    _tc_trig_body(wf_ref, b_ref, out_ref)


def _tc_trig_first(wf, bx):
    return pl.pallas_call(
        _tc_trig_body,
        grid=(BLKS_PER_STAGE,),
        in_specs=[
            pl.BlockSpec((ROWS_PER_BLK, 2 * D_HALF), lambda i: (i, 0)),
            pl.BlockSpec((ROWS_PER_BLK, 2 * D_HALF), lambda i: (i, 0)),
        ],
        out_specs=pl.BlockSpec((ROWS_PER_BLK, 2 * D_HALF), lambda i: (i, 0)),
        out_shape=jax.ShapeDtypeStruct((N, 2 * D_HALF), jnp.float32),
    )(wf, bx)


def _tc_trig_stage(stage, wf, bx, prev):
    return pl.pallas_call(
        _tc_trig_body_alias,
        grid=(BLKS_PER_STAGE,),
        in_specs=[
            pl.BlockSpec((ROWS_PER_BLK, 2 * D_HALF), lambda i: (i, 0)),
            pl.BlockSpec((ROWS_PER_BLK, 2 * D_HALF), lambda i: (i, 0)),
            pl.BlockSpec(memory_space=pl.ANY),
        ],
        out_specs=pl.BlockSpec(
            (ROWS_PER_BLK, 2 * D_HALF),
            lambda i, stage=stage: (i + stage * BLKS_PER_STAGE, 0)),
        out_shape=jax.ShapeDtypeStruct((N, 2 * D_HALF), jnp.float32),
        input_output_aliases={2: 0},
    )(wf, bx, prev)


def kernel(x, word_table, freq_table, phase_table):
    x_flat = x.reshape(N)
    wf_tbl = jnp.concatenate([word_table, freq_table], axis=1)
    halves = []
    for s in range(NSPLIT):
        xs = lax.slice_in_dim(x_flat, s * NH, (s + 1) * NH)
        halves.append(_sc_gather()(xs, wf_tbl, phase_table))
    out = _tc_trig_first(*halves[0])
    for s in range(1, NSPLIT):
        out = _tc_trig_stage(s, *halves[s], out)
    return out.reshape(B, L, 2 * D_HALF)


# speedup vs baseline: 2.9121x; 1.0647x over previous
import functools
import math

import jax
import jax.numpy as jnp
from jax import lax
from jax.experimental import pallas as pl
from jax.experimental.pallas import tpu as pltpu
from jax.experimental.pallas import tpu_sc as plsc

B, L = 4096, 200
D_HALF = 64
N = B * L
NSPLIT = 4
NH = N // NSPLIT
NC, NS = 2, 16
NW = NC * NS
PER_W = NH // NW
CHUNK = 128
N_CHUNKS = PER_W // CHUNK


def _sc_body(x_hbm, wf_hbm, phase_hbm, wf_out, b_out,
             idx_a, wf_a, b_a, idx_b, wf_b, b_b,
             sg_a, sw_a, sg_b, sw_b):
    wid = lax.axis_index("s") * NC + lax.axis_index("c")
    wbase = wid * PER_W

    sets = ((idx_a, wf_a, b_a, sg_a, sw_a),
            (idx_b, wf_b, b_b, sg_b, sw_b))

    def do_chunk(ci, idx_v, wf_v, b_v, sg, sw):
        base = wbase + ci * CHUNK
        @pl.when(ci >= 2)
        def _():
            pltpu.make_async_copy(
                wf_v, wf_out.at[pl.ds(base, CHUNK)], sw).wait()
            pltpu.make_async_copy(
                b_v, b_out.at[pl.ds(base, CHUNK), pl.ds(0, D_HALF)], sw).wait()

        pltpu.sync_copy(x_hbm.at[pl.ds(base, CHUNK)], idx_v)
        pltpu.async_copy(wf_hbm.at[idx_v], wf_v, sg)
        pltpu.async_copy(phase_hbm.at[idx_v], b_v, sg)
        pltpu.make_async_copy(wf_hbm.at[idx_v], wf_v, sg).wait()
        pltpu.make_async_copy(phase_hbm.at[idx_v], b_v, sg).wait()
        pltpu.async_copy(wf_v, wf_out.at[pl.ds(base, CHUNK)], sw)
        pltpu.async_copy(b_v, b_out.at[pl.ds(base, CHUNK), pl.ds(0, D_HALF)], sw)

    def pair_body(g, carry):
        do_chunk(2 * g, *sets[0])
        do_chunk(2 * g + 1, *sets[1])
        return carry

    lax.fori_loop(0, N_CHUNKS // 2, pair_body, 0, unroll=False)

    for (idx_v, wf_v, b_v, sg, sw) in sets:
        pltpu.make_async_copy(wf_v, wf_out.at[pl.ds(wbase, CHUNK)], sw).wait()
        pltpu.make_async_copy(
            b_v, b_out.at[pl.ds(wbase, CHUNK), pl.ds(0, D_HALF)], sw).wait()


@functools.cache
def _sc_gather():
    return pl.kernel(
        _sc_body,
        mesh=plsc.VectorSubcoreMesh(core_axis_name="c", subcore_axis_name="s"),
        compiler_params=pltpu.CompilerParams(use_tc_tiling_on_sc=False),
        out_type=[
            jax.ShapeDtypeStruct((NH, 2 * D_HALF), jnp.float32),
            jax.ShapeDtypeStruct((NH, 2 * D_HALF), jnp.float32),
        ],
        scratch_types=[
            pltpu.VMEM((CHUNK,), jnp.int32),
            pltpu.VMEM((CHUNK, 2 * D_HALF), jnp.float32),
            pltpu.VMEM((CHUNK, D_HALF), jnp.float32),
            pltpu.VMEM((CHUNK,), jnp.int32),
            pltpu.VMEM((CHUNK, 2 * D_HALF), jnp.float32),
            pltpu.VMEM((CHUNK, D_HALF), jnp.float32),
            pltpu.SemaphoreType.DMA, pltpu.SemaphoreType.DMA,
            pltpu.SemaphoreType.DMA, pltpu.SemaphoreType.DMA,
        ],
    )


ROWS_PER_BLK = 2048
BLKS_PER_STAGE = NH // ROWS_PER_BLK

_MAGIC = 12582912.0
_INV_PIO2 = 0.6366197723675814
_PIO2_HI = 1.57080078125
_PIO2_MID = -4.454455029158992e-06
_S1, _S2 = -0.16666667, 0.008332161
_C1, _C2, _C3 = -0.5, 0.041666418, -0.0013889048


def _sincos(ph):
    t = ph * _INV_PIO2
    n_big = t + _MAGIC
    nf = n_big - _MAGIC
    r = ph - nf * _PIO2_HI
    r = r - nf * _PIO2_MID
    r2 = r * r
    s = r + r * r2 * (_S1 + r2 * _S2)
    c = 1.0 + r2 * (_C1 + r2 * (_C2 + r2 * _C3))
    ni = lax.bitcast_convert_type(n_big, jnp.int32)
    swap = (ni & 1) == 1
    sinv = jnp.where(swap, c, s)
    cosv = jnp.where(swap, s, c)
    sgn_s = (ni & 2) << 30
    sgn_c = ((ni + 1) & 2) << 30
    sinv = lax.bitcast_convert_type(
        lax.bitcast_convert_type(sinv, jnp.int32) ^ sgn_s, jnp.float32)
    cosv = lax.bitcast_convert_type(
        lax.bitcast_convert_type(cosv, jnp.int32) ^ sgn_c, jnp.float32)
    return sinv, cosv


def _pos_col(i, sub):
    half = ROWS_PER_BLK // 2
    off = lax.rem(i * ROWS_PER_BLK + sub * half, L)
    v = lax.broadcasted_iota(jnp.int32, (half, 1), 0) + off
    q = (v * 5243) >> 20
    return (v - q * L + 1).astype(jnp.float32)


def _tc_trig_body(wf_ref, b_ref, out_ref):
    i = pl.program_id(0)
    half = ROWS_PER_BLK // 2
    lo, hi = pl.ds(0, half), pl.ds(half, half)
    ph_lo = _pos_col(i, 0) * wf_ref[lo, D_HALF:2 * D_HALF] + b_ref[lo, 0:D_HALF]
    ph_hi = _pos_col(i, 1) * wf_ref[hi, D_HALF:2 * D_HALF] + b_ref[hi, 0:D_HALF]
    ph = jnp.concatenate([ph_lo, ph_hi], axis=1)
    sinv, cosv = _sincos(ph)
    amp_lo = wf_ref[lo, 0:D_HALF]
    amp_hi = wf_ref[hi, 0:D_HALF]
    out_ref[lo, 0:D_HALF] = amp_lo * cosv[:, 0:D_HALF]
    out_ref[lo, D_HALF:2 * D_HALF] = amp_lo * sinv[:, 0:D_HALF]
    out_ref[hi, 0:D_HALF] = amp_hi * cosv[:, D_HALF:2 * D_HALF]
    out_ref[hi, D_HALF:2 * D_HALF] = amp_hi * sinv[:, D_HALF:2 * D_HALF]


def _tc_trig_body_alias(wf_ref, b_ref, _prev_ref, out_ref):
    _tc_trig_body(wf_ref, b_ref, out_ref)


def _tc_trig_first(wf, bx):
    return pl.pallas_call(
        _tc_trig_body,
        grid=(BLKS_PER_STAGE,),
        in_specs=[
            pl.BlockSpec((ROWS_PER_BLK, 2 * D_HALF), lambda i: (i, 0)),
            pl.BlockSpec((ROWS_PER_BLK, 2 * D_HALF), lambda i: (i, 0)),
        ],
        out_specs=pl.BlockSpec((ROWS_PER_BLK, 2 * D_HALF), lambda i: (i, 0)),
        out_shape=jax.ShapeDtypeStruct((N, 2 * D_HALF), jnp.float32),
    )(wf, bx)


def _tc_trig_stage(stage, wf, bx, prev):
    return pl.pallas_call(
        _tc_trig_body_alias,
        grid=(BLKS_PER_STAGE,),
        in_specs=[
            pl.BlockSpec((ROWS_PER_BLK, 2 * D_HALF), lambda i: (i, 0)),
            pl.BlockSpec((ROWS_PER_BLK, 2 * D_HALF), lambda i: (i, 0)),
            pl.BlockSpec(memory_space=pl.ANY),
        ],
        out_specs=pl.BlockSpec(
            (ROWS_PER_BLK, 2 * D_HALF),
            lambda i, stage=stage: (i + stage * BLKS_PER_STAGE, 0)),
        out_shape=jax.ShapeDtypeStruct((N, 2 * D_HALF), jnp.float32),
        input_output_aliases={2: 0},
    )(wf, bx, prev)


def kernel(x, word_table, freq_table, phase_table):
    x_flat = x.reshape(N)
    wf_tbl = jnp.concatenate([word_table, freq_table], axis=1)
    halves = []
    for s in range(NSPLIT):
        xs = lax.slice_in_dim(x_flat, s * NH, (s + 1) * NH)
        halves.append(_sc_gather()(xs, wf_tbl, phase_table))
    out = _tc_trig_first(*halves[0])
    for s in range(1, NSPLIT):
        out = _tc_trig_stage(s, *halves[s], out)
    return out.reshape(B, L, 2 * D_HALF)
